# Initial kernel scaffold; baseline (speedup 1.0000x reference)
import jax
import jax.numpy as jnp
from jax.experimental import pallas as pl


def kernel(s, v, dir_ij, Wij, senders, receivers, Wi1, bi1, Wi2, bi2, Wv, bv, Wm1, bm1, Wm2, bm2):
    raise NotImplementedError("write your pallas kernel here")


# speedup vs baseline: 3.9835x; 3.9835x over previous
import functools

import jax
import jax.numpy as jnp
from jax import lax
from jax.experimental import pallas as pl
from jax.experimental.pallas import tpu as pltpu
from jax.experimental.pallas import tpu_sc as plsc

H = 256
EPS = 1e-08

NC = 2
NS = 16
LANES = 16
NCHUNK = 8
CW = 32
B = 80


def _mlp_a_body(s_ref, w1_ref, b1_ref, w2_ref, b2_ref, o_ref):
    h = jnp.dot(s_ref[...], w1_ref[...], preferred_element_type=jnp.float32)
    h = h + b1_ref[...]
    h = h * jax.nn.sigmoid(h)
    o = jnp.dot(h, w2_ref[...], preferred_element_type=jnp.float32)
    o_ref[...] = o + b2_ref[...]


def _interaction(s2, Wi1, bi1, Wi2, bi2):
    n = s2.shape[0]
    r = 2000
    grid = n // r
    return pl.pallas_call(
        _mlp_a_body,
        grid=(grid,),
        in_specs=[
            pl.BlockSpec((r, H), lambda i: (i, 0)),
            pl.BlockSpec((H, H), lambda i: (0, 0)),
            pl.BlockSpec((1, H), lambda i: (0, 0)),
            pl.BlockSpec((H, 3 * H), lambda i: (0, 0)),
            pl.BlockSpec((1, 3 * H), lambda i: (0, 0)),
        ],
        out_specs=pl.BlockSpec((r, 3 * H), lambda i: (i, 0)),
        out_shape=jax.ShapeDtypeStruct((n, 3 * H), jnp.float32),
    )(s2, Wi1, bi1.reshape(1, H), Wi2, bi2.reshape(1, 3 * H))


def _edge_phase(x24, v24, wij, dir_ij, senders, receivers):
    n24 = x24.shape[0]
    n = n24 // 24
    e = senders.shape[0]
    nblk_per_tile = e // B // NS
    rows_per_tile = n // NS
    zrows = 25

    mesh = plsc.VectorSubcoreMesh(
        core_axis_name="c", subcore_axis_name="s",
        num_cores=NC, num_subcores=NS)

    @functools.partial(
        pl.kernel,
        out_type=(
            jax.ShapeDtypeStruct((n, NCHUNK, CW), jnp.float32),
            jax.ShapeDtypeStruct((n, 3, NCHUNK, CW), jnp.float32),
        ),
        mesh=mesh,
        scratch_types=[
            pltpu.VMEM_SHARED((n, 4 * CW), jnp.float32),
            pltpu.VMEM((zrows, 4 * CW), jnp.float32),
            pltpu.VMEM((B,), jnp.int32),
            pltpu.VMEM((B,), jnp.int32),
            pltpu.VMEM((B,), jnp.int32),
            pltpu.VMEM((B,), jnp.int32),
            pltpu.VMEM((B,), jnp.int32),
            pltpu.VMEM((B, CW), jnp.float32),
            pltpu.VMEM((B, CW), jnp.float32),
            pltpu.VMEM((B, CW), jnp.float32),
            pltpu.VMEM((B, CW), jnp.float32),
            pltpu.VMEM((B, CW), jnp.float32),
            pltpu.VMEM((B, CW), jnp.float32),
            pltpu.VMEM((B, CW), jnp.float32),
            pltpu.VMEM((B, CW), jnp.float32),
            pltpu.VMEM((B, CW), jnp.float32),
            pltpu.VMEM((B * 3 + LANES,), jnp.float32),
            pltpu.VMEM((B, 4 * CW), jnp.float32),
            pltpu.SemaphoreType.DMA,
        ],
        compiler_params=pltpu.CompilerParams(use_tc_tiling_on_sc=False),
    )
    def ek(x_hbm, v_hbm, w_hbm, dir_hbm, snd_hbm, rcv_hbm, ds_out, dv_out,
           accum, zbuf, sidx, ridx, g0, g1, g2,
           xds, xd1, xd2, vj0, vj1, vj2, wds, wd1, wd2,
           dirb, orows, sem):
        cid = lax.axis_index("c")
        tid = lax.axis_index("s")

        def zfill(i, _):
            zbuf[i // 8, pl.ds((i % 8) * LANES, LANES)] = jnp.zeros(
                (LANES,), jnp.float32)
            return 0
        lax.fori_loop(0, zrows * 8, zfill, 0)

        n0 = tid * rows_per_tile

        for ch_l in range(NCHUNK // NC):
            ch = cid * (NCHUNK // NC) + ch_l

            for z in range(rows_per_tile // zrows):
                pltpu.sync_copy(
                    zbuf, accum.at[pl.ds(n0 + z * zrows, zrows), :])
            plsc.subcore_barrier()

            def blk_body(bi, _):
                b = bi * NS + tid
                e0 = b * B
                pltpu.sync_copy(snd_hbm.at[pl.ds(e0, B)], sidx)
                pltpu.sync_copy(rcv_hbm.at[pl.ds(e0, B)], ridx)
                pltpu.sync_copy(dir_hbm.at[pl.ds(e0 * 3, B * 3)],
                                dirb.at[pl.ds(0, B * 3)])

                def idx_body(k, _):
                    sl = pl.ds(k * LANES, LANES)
                    r = ridx[sl] * 24 + ch
                    g0[sl] = r
                    g1[sl] = r + 8
                    g2[sl] = r + 16
                    return 0
                lax.fori_loop(0, B // LANES, idx_body, 0)

                c1 = pltpu.async_copy(x_hbm.at[g0], xds, sem)
                c2 = pltpu.async_copy(x_hbm.at[g1], xd1, sem)
                c3 = pltpu.async_copy(x_hbm.at[g2], xd2, sem)
                c4 = pltpu.async_copy(v_hbm.at[g0], vj0, sem)
                c5 = pltpu.async_copy(v_hbm.at[g1], vj1, sem)
                c6 = pltpu.async_copy(v_hbm.at[g2], vj2, sem)
                pltpu.sync_copy(w_hbm.at[pl.ds(e0, B), ch, :], wds)
                pltpu.sync_copy(w_hbm.at[pl.ds(e0, B), 8 + ch, :], wd1)
                pltpu.sync_copy(w_hbm.at[pl.ds(e0, B), 16 + ch, :], wd2)
                c1.wait(); c2.wait(); c3.wait()
                c4.wait(); c5.wait(); c6.wait()

                def e_body(ei, _):
                    d3 = dirb[pl.ds(ei * 3, LANES)]
                    dd0 = d3[0]
                    dd1 = d3[1]
                    dd2 = d3[2]
                    for j in range(CW // LANES):
                        sl = pl.ds(j * LANES, LANES)
                        a1 = xd1[ei, sl] * wd1[ei, sl]
                        a2 = xd2[ei, sl] * wd2[ei, sl]
                        orows[ei, pl.ds(j * LANES, LANES)] = (
                            xds[ei, sl] * wds[ei, sl])
                        orows[ei, pl.ds(CW + j * LANES, LANES)] = (
                            a1 * dd0 + a2 * vj0[ei, sl])
                        orows[ei, pl.ds(2 * CW + j * LANES, LANES)] = (
                            a1 * dd1 + a2 * vj1[ei, sl])
                        orows[ei, pl.ds(3 * CW + j * LANES, LANES)] = (
                            a1 * dd2 + a2 * vj2[ei, sl])
                    return 0
                lax.fori_loop(0, B, e_body, 0)

                pltpu.sync_copy(orows, accum.at[sidx], add=True)
                return 0
            lax.fori_loop(0, nblk_per_tile, blk_body, 0)
            plsc.subcore_barrier()

            pltpu.sync_copy(
                accum.at[pl.ds(n0, rows_per_tile), pl.ds(0, CW)],
                ds_out.at[pl.ds(n0, rows_per_tile), ch, :])
            for d in range(3):
                pltpu.sync_copy(
                    accum.at[pl.ds(n0, rows_per_tile),
                             pl.ds((d + 1) * CW, CW)],
                    dv_out.at[pl.ds(n0, rows_per_tile), d, ch, :])

    return ek(x24, v24, wij, dir_ij, senders, receivers)


def _mix_body(s_ref, v_ref, ds_ref, dv_ref, wv_ref, bv_ref,
              wm1a_ref, wm1b_ref, bm1_ref, wm2_ref, bm2_ref,
              so_ref, vo_ref):
    s1 = s_ref[...] + ds_ref[...]
    u0 = v_ref[:, 0, :] + dv_ref[:, 0, :]
    u1 = v_ref[:, 1, :] + dv_ref[:, 1, :]
    u2 = v_ref[:, 2, :] + dv_ref[:, 2, :]
    wv = wv_ref[...]
    bv = bv_ref[...]
    m0 = jnp.dot(u0, wv, preferred_element_type=jnp.float32) + bv
    m1 = jnp.dot(u1, wv, preferred_element_type=jnp.float32) + bv
    m2 = jnp.dot(u2, wv, preferred_element_type=jnp.float32) + bv
    l0, r0 = m0[:, :H], m0[:, H:]
    l1, r1 = m1[:, :H], m1[:, H:]
    l2, r2 = m2[:, :H], m2[:, H:]
    vnorm = jnp.sqrt(l0 * l0 + l1 * l1 + l2 * l2 + EPS)
    h = (jnp.dot(s1, wm1a_ref[...], preferred_element_type=jnp.float32)
         + jnp.dot(vnorm, wm1b_ref[...], preferred_element_type=jnp.float32)
         + bm1_ref[...])
    h = h * jax.nn.sigmoid(h)
    mix = jnp.dot(h, wm2_ref[...], preferred_element_type=jnp.float32)
    mix = mix + bm2_ref[...]
    ds2 = mix[:, :H]
    dvu = mix[:, H:2 * H]
    dsv = mix[:, 2 * H:]
    dot_lr = l0 * r0 + l1 * r1 + l2 * r2
    so_ref[...] = s1 + ds2 + dsv * dot_lr
    vo_ref[:, 0, :] = u0 + dvu * r0
    vo_ref[:, 1, :] = u1 + dvu * r1
    vo_ref[:, 2, :] = u2 + dvu * r2


def _mixing(s2, v, ds, dv, Wv, bv, Wm1, bm1, Wm2, bm2):
    n = s2.shape[0]
    r = 1000
    grid = n // r
    return pl.pallas_call(
        _mix_body,
        grid=(grid,),
        in_specs=[
            pl.BlockSpec((r, H), lambda i: (i, 0)),
            pl.BlockSpec((r, 3, H), lambda i: (i, 0, 0)),
            pl.BlockSpec((r, H), lambda i: (i, 0)),
            pl.BlockSpec((r, 3, H), lambda i: (i, 0, 0)),
            pl.BlockSpec((H, 2 * H), lambda i: (0, 0)),
            pl.BlockSpec((1, 2 * H), lambda i: (0, 0)),
            pl.BlockSpec((H, H), lambda i: (0, 0)),
            pl.BlockSpec((H, H), lambda i: (0, 0)),
            pl.BlockSpec((1, H), lambda i: (0, 0)),
            pl.BlockSpec((H, 3 * H), lambda i: (0, 0)),
            pl.BlockSpec((1, 3 * H), lambda i: (0, 0)),
        ],
        out_specs=[
            pl.BlockSpec((r, H), lambda i: (i, 0)),
            pl.BlockSpec((r, 3, H), lambda i: (i, 0, 0)),
        ],
        out_shape=[
            jax.ShapeDtypeStruct((n, H), jnp.float32),
            jax.ShapeDtypeStruct((n, 3, H), jnp.float32),
        ],
    )(s2, v, ds, dv, Wv, bv.reshape(1, 2 * H), Wm1[:H], Wm1[H:],
      bm1.reshape(1, H), Wm2, bm2.reshape(1, 3 * H))


def kernel(s, v, dir_ij, Wij, senders, receivers,
           Wi1, bi1, Wi2, bi2, Wv, bv, Wm1, bm1, Wm2, bm2):
    n = s.shape[0]
    e = senders.shape[0]
    s2 = s.reshape(n, H)
    x = _interaction(s2, Wi1, bi1, Wi2, bi2)
    x24 = x.reshape(n * 24, CW)
    v24 = v.reshape(n * 24, CW)
    wij = Wij.reshape(e, 24, CW)
    ds8, dv8 = _edge_phase(x24, v24, wij, dir_ij.reshape(e * 3),
                           senders, receivers)
    ds = ds8.reshape(n, H)
    dv = dv8.reshape(n, 3, H)
    so, vo = _mixing(s2, v, ds, dv, Wv, bv, Wm1, bm1, Wm2, bm2)
    return (so.reshape(n, 1, H), vo)
